# Initial kernel scaffold; baseline (speedup 1.0000x reference)
import jax
import jax.numpy as jnp
from jax.experimental import pallas as pl


def kernel(xyz, feature):
    raise NotImplementedError("write your pallas kernel here")


# speedup vs baseline: 26.9979x; 26.9979x over previous
import functools

import jax
import jax.numpy as jnp
from jax import lax
from jax.experimental import pallas as pl
from jax.experimental.pallas import tpu as pltpu
from jax.experimental.pallas import tpu_sc as plsc


_RATIO = 4


_TILE = 128


def _fps_body(x_ref, y_ref, z_ref, f0_ref, cent_ref, sx_ref, sy_ref, sz_ref):
    B, N = x_ref.shape
    M = cent_ref.shape[1]
    x = x_ref[...]
    y = y_ref[...]
    z = z_ref[...]
    f0 = f0_ref[...]
    iota_n = lax.broadcasted_iota(jnp.int32, (B, N), 1)
    iota_t = lax.broadcasted_iota(jnp.int32, (B, _TILE), 1)
    d0 = jnp.full((B, N), 1e10, dtype=jnp.float32)

    def inner(j, carry):
        d, f, tc, tx, ty, tz = carry
        mask = iota_n == f
        cx = jnp.sum(jnp.where(mask, x, 0.0), axis=1, keepdims=True)
        cy = jnp.sum(jnp.where(mask, y, 0.0), axis=1, keepdims=True)
        cz = jnp.sum(jnp.where(mask, z, 0.0), axis=1, keepdims=True)
        sel = iota_t == j
        tc = jnp.where(sel, f, tc)
        tx = jnp.where(sel, cx, tx)
        ty = jnp.where(sel, cy, ty)
        tz = jnp.where(sel, cz, tz)
        dx = x - cx
        dy = y - cy
        dz = z - cz
        dist = dx * dx + dy * dy + dz * dz
        dnew = jnp.minimum(d, dist)
        m = jnp.max(dnew, axis=1, keepdims=True)
        fnew = jnp.min(jnp.where(dnew == m, iota_n, jnp.int32(N)),
                       axis=1, keepdims=True)
        return dnew, fnew, tc, tx, ty, tz

    def outer(k, carry):
        d, f = carry
        zc = jnp.zeros((B, _TILE), jnp.int32)
        zf = jnp.zeros((B, _TILE), jnp.float32)
        d, f, tc, tx, ty, tz = lax.fori_loop(
            0, _TILE, inner, (d, f, zc, zf, zf, zf))
        base = pl.multiple_of(k * _TILE, _TILE)
        cent_ref[:, pl.ds(base, _TILE)] = tc
        sx_ref[:, pl.ds(base, _TILE)] = tx
        sy_ref[:, pl.ds(base, _TILE)] = ty
        sz_ref[:, pl.ds(base, _TILE)] = tz
        return d, f

    lax.fori_loop(0, M // _TILE, outer, (d0, f0))


def _run_fps(x, y, z, f0):
    B, N = x.shape
    M = N // _RATIO
    return pl.pallas_call(
        _fps_body,
        out_shape=(
            jax.ShapeDtypeStruct((B, M), jnp.int32),
            jax.ShapeDtypeStruct((B, M), jnp.float32),
            jax.ShapeDtypeStruct((B, M), jnp.float32),
            jax.ShapeDtypeStruct((B, M), jnp.float32),
        ),
    )(x, y, z, f0)


def _sc_gather(table, idx2, n_rows, feat_dim):
    nw = 32
    rows_per_w = n_rows // nw
    chunks = rows_per_w // 128
    mesh = plsc.VectorSubcoreMesh(core_axis_name="c", subcore_axis_name="s")

    @functools.partial(
        pl.kernel,
        mesh=mesh,
        out_type=jax.ShapeDtypeStruct((n_rows, feat_dim), jnp.float32),
        scratch_types=[
            pltpu.VMEM((chunks, 128), jnp.int32),
            pltpu.VMEM((rows_per_w, feat_dim), jnp.float32),
            pltpu.SemaphoreType.DMA,
        ],
    )
    def gather_kernel(table_hbm, idx_hbm, out_hbm, idx_v, rows_v, sem):
        wid = lax.axis_index("c") * 16 + lax.axis_index("s")
        pltpu.sync_copy(idx_hbm.at[pl.ds(wid * chunks, chunks)], idx_v)
        copies = [
            pltpu.async_copy(table_hbm.at[idx_v.at[j]],
                             rows_v.at[pl.ds(j * 128, 128)], sem)
            for j in range(chunks)
        ]
        for cp in copies:
            cp.wait()
        pltpu.sync_copy(rows_v, out_hbm.at[pl.ds(wid * rows_per_w, rows_per_w)])

    return gather_kernel(table, idx2)


def kernel(xyz, feature):
    B, _, N = xyz.shape
    Cf = feature.shape[1]
    M = N // _RATIO

    fkey = jax.random.key(1)
    f0 = jax.random.randint(fkey, (B,), 0, N).astype(jnp.int32).reshape(B, 1)

    x = xyz[:, 0, :]
    y = xyz[:, 1, :]
    z = xyz[:, 2, :]
    cent, sx, sy, sz = _run_fps(x, y, z, f0)
    sampled_xyz = jnp.stack([sx, sy, sz], axis=1)

    table = feature.transpose(0, 2, 1).reshape(B * N, Cf)
    flat_idx = (cent + jnp.arange(B, dtype=jnp.int32)[:, None] * N)
    idx2 = flat_idx.reshape((B * M) // 128, 128)
    gathered = _sc_gather(table, idx2, B * M, Cf)
    sampled_feature = gathered.reshape(B, M, Cf).transpose(0, 2, 1)

    return sampled_xyz, sampled_feature


# speedup vs baseline: 34.0515x; 1.2613x over previous
import functools

import jax
import jax.numpy as jnp
from jax import lax
from jax.experimental import pallas as pl
from jax.experimental.pallas import tpu as pltpu
from jax.experimental.pallas import tpu_sc as plsc


_RATIO = 4


_TILE = 128
_W = 1024


def _fps_body(x_ref, y_ref, z_ref, f0_ref, cent_ref, sx_ref, sy_ref, sz_ref,
              d_ref):
    B, N = x_ref.shape
    M = cent_ref.shape[1]
    nblk = N // _W
    f0 = f0_ref[...]
    iota_t = lax.broadcasted_iota(jnp.int32, (B, _TILE), 1)
    iota_w = lax.broadcasted_iota(jnp.int32, (B, _W), 1)
    sent = jnp.full((B, 1), N, jnp.int32)
    d_ref[...] = jnp.full((B, N), 1e10, dtype=jnp.float32)

    def coords_of(f):
        cx = jnp.zeros((B, 1), jnp.float32)
        cy = jnp.zeros((B, 1), jnp.float32)
        cz = jnp.zeros((B, 1), jnp.float32)
        for k in range(nblk):
            sl = pl.ds(k * _W, _W)
            mb = (iota_w + (k * _W)) == f
            cx += jnp.sum(jnp.where(mb, x_ref[:, sl], 0.0), axis=1,
                          keepdims=True)
            cy += jnp.sum(jnp.where(mb, y_ref[:, sl], 0.0), axis=1,
                          keepdims=True)
            cz += jnp.sum(jnp.where(mb, z_ref[:, sl], 0.0), axis=1,
                          keepdims=True)
        return cx, cy, cz

    def inner(j, carry):
        f, cx, cy, cz, tc, tx, ty, tz = carry
        sel = iota_t == j
        tc = jnp.where(sel, f, tc)
        tx = jnp.where(sel, cx, tx)
        ty = jnp.where(sel, cy, ty)
        tz = jnp.where(sel, cz, tz)
        macc = jnp.full((B, _W), -1.0, jnp.float32)
        for k in range(nblk):
            sl = pl.ds(k * _W, _W)
            dx = x_ref[:, sl] - cx
            dy = y_ref[:, sl] - cy
            dz = z_ref[:, sl] - cz
            dist = dx * dx + dy * dy + dz * dz
            db = jnp.minimum(d_ref[:, sl], dist)
            d_ref[:, sl] = db
            macc = jnp.maximum(macc, db)
        m = jnp.max(macc, axis=1, keepdims=True)
        bi = sent
        bx = jnp.zeros((B, 1), jnp.float32)
        by = jnp.zeros((B, 1), jnp.float32)
        bz = jnp.zeros((B, 1), jnp.float32)
        for k in range(nblk):
            sl = pl.ds(k * _W, _W)
            iob = iota_w + (k * _W)
            selb = d_ref[:, sl] == m
            idxb = jnp.min(jnp.where(selb, iob, N), axis=1, keepdims=True)
            mb = iob == idxb
            cxb = jnp.sum(jnp.where(mb, x_ref[:, sl], 0.0), axis=1,
                          keepdims=True)
            cyb = jnp.sum(jnp.where(mb, y_ref[:, sl], 0.0), axis=1,
                          keepdims=True)
            czb = jnp.sum(jnp.where(mb, z_ref[:, sl], 0.0), axis=1,
                          keepdims=True)
            better = idxb < bi
            bi = jnp.where(better, idxb, bi)
            bx = jnp.where(better, cxb, bx)
            by = jnp.where(better, cyb, by)
            bz = jnp.where(better, czb, bz)
        return bi, bx, by, bz, tc, tx, ty, tz

    def outer(k, carry):
        f, cx, cy, cz = carry
        zc = jnp.zeros((B, _TILE), jnp.int32)
        zf = jnp.zeros((B, _TILE), jnp.float32)
        f, cx, cy, cz, tc, tx, ty, tz = lax.fori_loop(
            0, _TILE, inner, (f, cx, cy, cz, zc, zf, zf, zf))
        base = pl.multiple_of(k * _TILE, _TILE)
        cent_ref[:, pl.ds(base, _TILE)] = tc
        sx_ref[:, pl.ds(base, _TILE)] = tx
        sy_ref[:, pl.ds(base, _TILE)] = ty
        sz_ref[:, pl.ds(base, _TILE)] = tz
        return f, cx, cy, cz

    cx0, cy0, cz0 = coords_of(f0)
    lax.fori_loop(0, M // _TILE, outer, (f0, cx0, cy0, cz0))


def _run_fps(x, y, z, f0):
    B, N = x.shape
    M = N // _RATIO
    return pl.pallas_call(
        _fps_body,
        out_shape=(
            jax.ShapeDtypeStruct((B, M), jnp.int32),
            jax.ShapeDtypeStruct((B, M), jnp.float32),
            jax.ShapeDtypeStruct((B, M), jnp.float32),
            jax.ShapeDtypeStruct((B, M), jnp.float32),
        ),
        scratch_shapes=[pltpu.VMEM((B, N), jnp.float32)],
    )(x, y, z, f0)


def _sc_gather(table, idx2, n_rows, feat_dim):
    nw = 32
    rows_per_w = n_rows // nw
    chunks = rows_per_w // 128
    mesh = plsc.VectorSubcoreMesh(core_axis_name="c", subcore_axis_name="s")

    @functools.partial(
        pl.kernel,
        mesh=mesh,
        out_type=jax.ShapeDtypeStruct((n_rows, feat_dim), jnp.float32),
        scratch_types=[
            pltpu.VMEM((chunks, 128), jnp.int32),
            pltpu.VMEM((rows_per_w, feat_dim), jnp.float32),
            pltpu.SemaphoreType.DMA,
        ],
    )
    def gather_kernel(table_hbm, idx_hbm, out_hbm, idx_v, rows_v, sem):
        wid = lax.axis_index("c") * 16 + lax.axis_index("s")
        pltpu.sync_copy(idx_hbm.at[pl.ds(wid * chunks, chunks)], idx_v)
        copies = [
            pltpu.async_copy(table_hbm.at[idx_v.at[j]],
                             rows_v.at[pl.ds(j * 128, 128)], sem)
            for j in range(chunks)
        ]
        for cp in copies:
            cp.wait()
        pltpu.sync_copy(rows_v, out_hbm.at[pl.ds(wid * rows_per_w, rows_per_w)])

    return gather_kernel(table, idx2)


def kernel(xyz, feature):
    B, _, N = xyz.shape
    Cf = feature.shape[1]
    M = N // _RATIO

    fkey = jax.random.key(1)
    f0 = jax.random.randint(fkey, (B,), 0, N).astype(jnp.int32).reshape(B, 1)

    x = xyz[:, 0, :]
    y = xyz[:, 1, :]
    z = xyz[:, 2, :]
    cent, sx, sy, sz = _run_fps(x, y, z, f0)
    sampled_xyz = jnp.stack([sx, sy, sz], axis=1)

    table = feature.transpose(0, 2, 1).reshape(B * N, Cf)
    flat_idx = (cent + jnp.arange(B, dtype=jnp.int32)[:, None] * N)
    idx2 = flat_idx.reshape((B * M) // 128, 128)
    gathered = _sc_gather(table, idx2, B * M, Cf)
    sampled_feature = gathered.reshape(B, M, Cf).transpose(0, 2, 1)

    return sampled_xyz, sampled_feature


# speedup vs baseline: 38.9527x; 1.1439x over previous
import functools

import jax
import jax.numpy as jnp
from jax import lax
from jax.experimental import pallas as pl
from jax.experimental.pallas import tpu as pltpu
from jax.experimental.pallas import tpu_sc as plsc


_RATIO = 4


_TILE = 128
_W = 1024


def _fps_body(x_ref, y_ref, z_ref, f0_ref, cent_ref, sx_ref, sy_ref, sz_ref,
              d_ref):
    B, N = x_ref.shape
    M = cent_ref.shape[1]
    nblk = N // _W
    f0 = f0_ref[...]
    iota_t = lax.broadcasted_iota(jnp.int32, (B, _TILE), 1)
    iota_w = lax.broadcasted_iota(jnp.int32, (B, _W), 1)
    sent = jnp.full((B, 1), N, jnp.int32)
    d_ref[...] = jnp.full((B, N), 1e10, dtype=jnp.float32)

    def coords_of(f):
        cx = jnp.zeros((B, 1), jnp.float32)
        cy = jnp.zeros((B, 1), jnp.float32)
        cz = jnp.zeros((B, 1), jnp.float32)
        for k in range(nblk):
            sl = pl.ds(k * _W, _W)
            mb = (iota_w + (k * _W)) == f
            cx += jnp.sum(jnp.where(mb, x_ref[:, sl], 0.0), axis=1,
                          keepdims=True)
            cy += jnp.sum(jnp.where(mb, y_ref[:, sl], 0.0), axis=1,
                          keepdims=True)
            cz += jnp.sum(jnp.where(mb, z_ref[:, sl], 0.0), axis=1,
                          keepdims=True)
        return cx, cy, cz

    def inner(j, carry):
        f, cx, cy, cz, tc, tx, ty, tz = carry
        sel = iota_t == j
        tc = jnp.where(sel, f, tc)
        tx = jnp.where(sel, cx, tx)
        ty = jnp.where(sel, cy, ty)
        tz = jnp.where(sel, cz, tz)
        best_m = jnp.full((B, 1), -1.0, jnp.float32)
        bi = sent
        bx = jnp.zeros((B, 1), jnp.float32)
        by = jnp.zeros((B, 1), jnp.float32)
        bz = jnp.zeros((B, 1), jnp.float32)
        for k in range(nblk):
            sl = pl.ds(k * _W, _W)
            iob = iota_w + (k * _W)
            xb = x_ref[:, sl]
            yb = y_ref[:, sl]
            zb = z_ref[:, sl]
            dx = xb - cx
            dy = yb - cy
            dz = zb - cz
            dist = dx * dx + dy * dy + dz * dz
            db = jnp.minimum(d_ref[:, sl], dist)
            d_ref[:, sl] = db
            bm = jnp.max(db, axis=1, keepdims=True)
            selb = db == bm
            idxb = jnp.min(jnp.where(selb, iob, N), axis=1, keepdims=True)
            mb = iob == idxb
            cxb = jnp.sum(jnp.where(mb, xb, 0.0), axis=1, keepdims=True)
            cyb = jnp.sum(jnp.where(mb, yb, 0.0), axis=1, keepdims=True)
            czb = jnp.sum(jnp.where(mb, zb, 0.0), axis=1, keepdims=True)
            better = bm > best_m
            best_m = jnp.where(better, bm, best_m)
            bi = jnp.where(better, idxb, bi)
            bx = jnp.where(better, cxb, bx)
            by = jnp.where(better, cyb, by)
            bz = jnp.where(better, czb, bz)
        return bi, bx, by, bz, tc, tx, ty, tz

    def outer(k, carry):
        f, cx, cy, cz = carry
        zc = jnp.zeros((B, _TILE), jnp.int32)
        zf = jnp.zeros((B, _TILE), jnp.float32)
        f, cx, cy, cz, tc, tx, ty, tz = lax.fori_loop(
            0, _TILE, inner, (f, cx, cy, cz, zc, zf, zf, zf))
        base = pl.multiple_of(k * _TILE, _TILE)
        cent_ref[:, pl.ds(base, _TILE)] = tc
        sx_ref[:, pl.ds(base, _TILE)] = tx
        sy_ref[:, pl.ds(base, _TILE)] = ty
        sz_ref[:, pl.ds(base, _TILE)] = tz
        return f, cx, cy, cz

    cx0, cy0, cz0 = coords_of(f0)
    lax.fori_loop(0, M // _TILE, outer, (f0, cx0, cy0, cz0))


def _run_fps(x, y, z, f0):
    B, N = x.shape
    M = N // _RATIO
    return pl.pallas_call(
        _fps_body,
        out_shape=(
            jax.ShapeDtypeStruct((B, M), jnp.int32),
            jax.ShapeDtypeStruct((B, M), jnp.float32),
            jax.ShapeDtypeStruct((B, M), jnp.float32),
            jax.ShapeDtypeStruct((B, M), jnp.float32),
        ),
        scratch_shapes=[pltpu.VMEM((B, N), jnp.float32)],
    )(x, y, z, f0)


def _sc_gather(table, idx2, n_rows, feat_dim):
    nw = 32
    rows_per_w = n_rows // nw
    chunks = rows_per_w // 128
    mesh = plsc.VectorSubcoreMesh(core_axis_name="c", subcore_axis_name="s")

    @functools.partial(
        pl.kernel,
        mesh=mesh,
        out_type=jax.ShapeDtypeStruct((n_rows, feat_dim), jnp.float32),
        scratch_types=[
            pltpu.VMEM((chunks, 128), jnp.int32),
            pltpu.VMEM((rows_per_w, feat_dim), jnp.float32),
            pltpu.SemaphoreType.DMA,
        ],
    )
    def gather_kernel(table_hbm, idx_hbm, out_hbm, idx_v, rows_v, sem):
        wid = lax.axis_index("c") * 16 + lax.axis_index("s")
        pltpu.sync_copy(idx_hbm.at[pl.ds(wid * chunks, chunks)], idx_v)
        copies = [
            pltpu.async_copy(table_hbm.at[idx_v.at[j]],
                             rows_v.at[pl.ds(j * 128, 128)], sem)
            for j in range(chunks)
        ]
        for cp in copies:
            cp.wait()
        pltpu.sync_copy(rows_v, out_hbm.at[pl.ds(wid * rows_per_w, rows_per_w)])

    return gather_kernel(table, idx2)


def kernel(xyz, feature):
    B, _, N = xyz.shape
    Cf = feature.shape[1]
    M = N // _RATIO

    fkey = jax.random.key(1)
    f0 = jax.random.randint(fkey, (B,), 0, N).astype(jnp.int32).reshape(B, 1)

    x = xyz[:, 0, :]
    y = xyz[:, 1, :]
    z = xyz[:, 2, :]
    cent, sx, sy, sz = _run_fps(x, y, z, f0)
    sampled_xyz = jnp.stack([sx, sy, sz], axis=1)

    table = feature.transpose(0, 2, 1).reshape(B * N, Cf)
    flat_idx = (cent + jnp.arange(B, dtype=jnp.int32)[:, None] * N)
    idx2 = flat_idx.reshape((B * M) // 128, 128)
    gathered = _sc_gather(table, idx2, B * M, Cf)
    sampled_feature = gathered.reshape(B, M, Cf).transpose(0, 2, 1)

    return sampled_xyz, sampled_feature


# speedup vs baseline: 39.8078x; 1.0220x over previous
import functools

import jax
import jax.numpy as jnp
from jax import lax
from jax.experimental import pallas as pl
from jax.experimental.pallas import tpu as pltpu
from jax.experimental.pallas import tpu_sc as plsc


_RATIO = 4


_TILE = 128
_W = 1024


def _fps_body(x_ref, y_ref, z_ref, f0_ref, cent_ref, sx_ref, sy_ref, sz_ref,
              d_ref):
    B, N = x_ref.shape
    M = cent_ref.shape[1]
    nblk = N // _W
    f0 = f0_ref[...]
    iota_t = lax.broadcasted_iota(jnp.int32, (B, _TILE), 1)
    iota_w = lax.broadcasted_iota(jnp.int32, (B, _W), 1)
    sent = jnp.full((B, 1), N, jnp.int32)
    d_ref[...] = jnp.full((B, N), 1e10, dtype=jnp.float32)

    def coords_of(f):
        cx = jnp.zeros((B, 1), jnp.float32)
        cy = jnp.zeros((B, 1), jnp.float32)
        cz = jnp.zeros((B, 1), jnp.float32)
        for k in range(nblk):
            sl = pl.ds(k * _W, _W)
            mb = (iota_w + (k * _W)) == f
            cx += jnp.sum(jnp.where(mb, x_ref[:, sl], 0.0), axis=1,
                          keepdims=True)
            cy += jnp.sum(jnp.where(mb, y_ref[:, sl], 0.0), axis=1,
                          keepdims=True)
            cz += jnp.sum(jnp.where(mb, z_ref[:, sl], 0.0), axis=1,
                          keepdims=True)
        return cx, cy, cz

    def inner(j, carry):
        f, cx, cy, cz, tc, tx, ty, tz = carry
        sel = iota_t == j
        tc = jnp.where(sel, f, tc)
        tx = jnp.where(sel, cx, tx)
        ty = jnp.where(sel, cy, ty)
        tz = jnp.where(sel, cz, tz)
        best_m = jnp.full((B, 1), -1.0, jnp.float32)
        bi = sent
        bx = jnp.zeros((B, 1), jnp.float32)
        by = jnp.zeros((B, 1), jnp.float32)
        bz = jnp.zeros((B, 1), jnp.float32)
        for k in range(nblk):
            sl = pl.ds(k * _W, _W)
            xb = x_ref[:, sl]
            yb = y_ref[:, sl]
            zb = z_ref[:, sl]
            dx = xb - cx
            dy = yb - cy
            dz = zb - cz
            dist = (dx * dx + dz * dz) + dy * dy
            db = jnp.minimum(d_ref[:, sl], dist)
            d_ref[:, sl] = db
            bm = jnp.max(db, axis=1, keepdims=True)
            selb = db == bm
            idxb = jnp.min(jnp.where(selb, iota_w, _W), axis=1, keepdims=True)
            mb = iota_w == idxb
            cxb = jnp.sum(jnp.where(mb, xb, 0.0), axis=1, keepdims=True)
            cyb = jnp.sum(jnp.where(mb, yb, 0.0), axis=1, keepdims=True)
            czb = jnp.sum(jnp.where(mb, zb, 0.0), axis=1, keepdims=True)
            better = bm > best_m
            best_m = jnp.where(better, bm, best_m)
            bi = jnp.where(better, idxb + (k * _W), bi)
            bx = jnp.where(better, cxb, bx)
            by = jnp.where(better, cyb, by)
            bz = jnp.where(better, czb, bz)
        return bi, bx, by, bz, tc, tx, ty, tz

    def outer(k, carry):
        f, cx, cy, cz = carry
        zc = jnp.zeros((B, _TILE), jnp.int32)
        zf = jnp.zeros((B, _TILE), jnp.float32)
        f, cx, cy, cz, tc, tx, ty, tz = lax.fori_loop(
            0, _TILE, inner, (f, cx, cy, cz, zc, zf, zf, zf), unroll=2)
        base = pl.multiple_of(k * _TILE, _TILE)
        cent_ref[:, pl.ds(base, _TILE)] = tc
        sx_ref[:, pl.ds(base, _TILE)] = tx
        sy_ref[:, pl.ds(base, _TILE)] = ty
        sz_ref[:, pl.ds(base, _TILE)] = tz
        return f, cx, cy, cz

    cx0, cy0, cz0 = coords_of(f0)
    lax.fori_loop(0, M // _TILE, outer, (f0, cx0, cy0, cz0))


def _run_fps(x, y, z, f0):
    B, N = x.shape
    M = N // _RATIO
    return pl.pallas_call(
        _fps_body,
        out_shape=(
            jax.ShapeDtypeStruct((B, M), jnp.int32),
            jax.ShapeDtypeStruct((B, M), jnp.float32),
            jax.ShapeDtypeStruct((B, M), jnp.float32),
            jax.ShapeDtypeStruct((B, M), jnp.float32),
        ),
        scratch_shapes=[pltpu.VMEM((B, N), jnp.float32)],
    )(x, y, z, f0)


def _sc_gather(table, idx2, n_rows, feat_dim):
    nw = 32
    rows_per_w = n_rows // nw
    chunks = rows_per_w // 128
    mesh = plsc.VectorSubcoreMesh(core_axis_name="c", subcore_axis_name="s")

    @functools.partial(
        pl.kernel,
        mesh=mesh,
        out_type=jax.ShapeDtypeStruct((n_rows, feat_dim), jnp.float32),
        scratch_types=[
            pltpu.VMEM((chunks, 128), jnp.int32),
            pltpu.VMEM((rows_per_w, feat_dim), jnp.float32),
            pltpu.SemaphoreType.DMA,
        ],
    )
    def gather_kernel(table_hbm, idx_hbm, out_hbm, idx_v, rows_v, sem):
        wid = lax.axis_index("c") * 16 + lax.axis_index("s")
        pltpu.sync_copy(idx_hbm.at[pl.ds(wid * chunks, chunks)], idx_v)
        copies = [
            pltpu.async_copy(table_hbm.at[idx_v.at[j]],
                             rows_v.at[pl.ds(j * 128, 128)], sem)
            for j in range(chunks)
        ]
        for cp in copies:
            cp.wait()
        pltpu.sync_copy(rows_v, out_hbm.at[pl.ds(wid * rows_per_w, rows_per_w)])

    return gather_kernel(table, idx2)


def kernel(xyz, feature):
    B, _, N = xyz.shape
    Cf = feature.shape[1]
    M = N // _RATIO

    fkey = jax.random.key(1)
    f0 = jax.random.randint(fkey, (B,), 0, N).astype(jnp.int32).reshape(B, 1)

    x = xyz[:, 0, :]
    y = xyz[:, 1, :]
    z = xyz[:, 2, :]
    cent, sx, sy, sz = _run_fps(x, y, z, f0)
    sampled_xyz = jnp.stack([sx, sy, sz], axis=1)

    table = feature.transpose(0, 2, 1).reshape(B * N, Cf)
    flat_idx = (cent + jnp.arange(B, dtype=jnp.int32)[:, None] * N)
    idx2 = flat_idx.reshape((B * M) // 128, 128)
    gathered = _sc_gather(table, idx2, B * M, Cf)
    sampled_feature = gathered.reshape(B, M, Cf).transpose(0, 2, 1)

    return sampled_xyz, sampled_feature


# speedup vs baseline: 40.5487x; 1.0186x over previous
import functools

import jax
import jax.numpy as jnp
from jax import lax
from jax.experimental import pallas as pl
from jax.experimental.pallas import tpu as pltpu
from jax.experimental.pallas import tpu_sc as plsc


_RATIO = 4


_TILE = 128
_W = 1024


def _fps_body(x_ref, y_ref, z_ref, f0_ref, cent_ref, sx_ref, sy_ref, sz_ref,
              d_ref):
    B, N = x_ref.shape
    M = cent_ref.shape[1]
    nblk = N // _W
    f0 = f0_ref[...]
    iota_t = lax.broadcasted_iota(jnp.int32, (B, _TILE), 1)
    iota_w = lax.broadcasted_iota(jnp.int32, (B, _W), 1)
    sent = jnp.full((B, 1), N, jnp.int32)
    d_ref[...] = jnp.full((B, N), 1e10, dtype=jnp.float32)

    def coords_of(f):
        cx = jnp.zeros((B, 1), jnp.float32)
        cy = jnp.zeros((B, 1), jnp.float32)
        cz = jnp.zeros((B, 1), jnp.float32)
        for k in range(nblk):
            sl = pl.ds(k * _W, _W)
            mb = (iota_w + (k * _W)) == f
            cx += jnp.sum(jnp.where(mb, x_ref[:, sl], 0.0), axis=1,
                          keepdims=True)
            cy += jnp.sum(jnp.where(mb, y_ref[:, sl], 0.0), axis=1,
                          keepdims=True)
            cz += jnp.sum(jnp.where(mb, z_ref[:, sl], 0.0), axis=1,
                          keepdims=True)
        return cx, cy, cz

    def inner(j, carry):
        f, cx, cy, cz, tc, tx, ty, tz = carry
        sel = iota_t == j
        tc = jnp.where(sel, f, tc)
        tx = jnp.where(sel, cx, tx)
        ty = jnp.where(sel, cy, ty)
        tz = jnp.where(sel, cz, tz)
        best_m = jnp.full((B, 1), -1.0, jnp.float32)
        bi = sent
        bx = jnp.zeros((B, 1), jnp.float32)
        by = jnp.zeros((B, 1), jnp.float32)
        bz = jnp.zeros((B, 1), jnp.float32)
        for k in range(nblk):
            sl = pl.ds(k * _W, _W)
            xb = x_ref[:, sl]
            yb = y_ref[:, sl]
            zb = z_ref[:, sl]
            dx = xb - cx
            dy = yb - cy
            dz = zb - cz
            dist = (dx * dx + dz * dz) + dy * dy
            db = jnp.minimum(d_ref[:, sl], dist)
            d_ref[:, sl] = db
            bm = jnp.max(db, axis=1, keepdims=True)
            selb = db == bm
            idxb = jnp.min(jnp.where(selb, iota_w, _W), axis=1, keepdims=True)
            mb = iota_w == idxb
            cxb = jnp.sum(jnp.where(mb, xb, 0.0), axis=1, keepdims=True)
            cyb = jnp.sum(jnp.where(mb, yb, 0.0), axis=1, keepdims=True)
            czb = jnp.sum(jnp.where(mb, zb, 0.0), axis=1, keepdims=True)
            better = bm > best_m
            best_m = jnp.where(better, bm, best_m)
            bi = jnp.where(better, idxb + (k * _W), bi)
            bx = jnp.where(better, cxb, bx)
            by = jnp.where(better, cyb, by)
            bz = jnp.where(better, czb, bz)
        return bi, bx, by, bz, tc, tx, ty, tz

    def outer(k, carry):
        f, cx, cy, cz = carry
        zc = jnp.zeros((B, _TILE), jnp.int32)
        zf = jnp.zeros((B, _TILE), jnp.float32)
        f, cx, cy, cz, tc, tx, ty, tz = lax.fori_loop(
            0, _TILE, inner, (f, cx, cy, cz, zc, zf, zf, zf), unroll=4)
        base = pl.multiple_of(k * _TILE, _TILE)
        cent_ref[:, pl.ds(base, _TILE)] = tc
        sx_ref[:, pl.ds(base, _TILE)] = tx
        sy_ref[:, pl.ds(base, _TILE)] = ty
        sz_ref[:, pl.ds(base, _TILE)] = tz
        return f, cx, cy, cz

    cx0, cy0, cz0 = coords_of(f0)
    lax.fori_loop(0, M // _TILE, outer, (f0, cx0, cy0, cz0))


def _run_fps(x, y, z, f0):
    B, N = x.shape
    M = N // _RATIO
    return pl.pallas_call(
        _fps_body,
        out_shape=(
            jax.ShapeDtypeStruct((B, M), jnp.int32),
            jax.ShapeDtypeStruct((B, M), jnp.float32),
            jax.ShapeDtypeStruct((B, M), jnp.float32),
            jax.ShapeDtypeStruct((B, M), jnp.float32),
        ),
        scratch_shapes=[pltpu.VMEM((B, N), jnp.float32)],
    )(x, y, z, f0)


def _sc_gather(table, idx2, n_rows, feat_dim):
    nw = 32
    rows_per_w = n_rows // nw
    chunks = rows_per_w // 128
    mesh = plsc.VectorSubcoreMesh(core_axis_name="c", subcore_axis_name="s")

    @functools.partial(
        pl.kernel,
        mesh=mesh,
        out_type=jax.ShapeDtypeStruct((n_rows, feat_dim), jnp.float32),
        scratch_types=[
            pltpu.VMEM((chunks, 128), jnp.int32),
            pltpu.VMEM((rows_per_w, feat_dim), jnp.float32),
            pltpu.SemaphoreType.DMA,
        ],
    )
    def gather_kernel(table_hbm, idx_hbm, out_hbm, idx_v, rows_v, sem):
        wid = lax.axis_index("c") * 16 + lax.axis_index("s")
        pltpu.sync_copy(idx_hbm.at[pl.ds(wid * chunks, chunks)], idx_v)
        copies = [
            pltpu.async_copy(table_hbm.at[idx_v.at[j]],
                             rows_v.at[pl.ds(j * 128, 128)], sem)
            for j in range(chunks)
        ]
        for cp in copies:
            cp.wait()
        pltpu.sync_copy(rows_v, out_hbm.at[pl.ds(wid * rows_per_w, rows_per_w)])

    return gather_kernel(table, idx2)


def kernel(xyz, feature):
    B, _, N = xyz.shape
    Cf = feature.shape[1]
    M = N // _RATIO

    fkey = jax.random.key(1)
    f0 = jax.random.randint(fkey, (B,), 0, N).astype(jnp.int32).reshape(B, 1)

    x = xyz[:, 0, :]
    y = xyz[:, 1, :]
    z = xyz[:, 2, :]
    cent, sx, sy, sz = _run_fps(x, y, z, f0)
    sampled_xyz = jnp.stack([sx, sy, sz], axis=1)

    table = feature.transpose(0, 2, 1).reshape(B * N, Cf)
    flat_idx = (cent + jnp.arange(B, dtype=jnp.int32)[:, None] * N)
    idx2 = flat_idx.reshape((B * M) // 128, 128)
    gathered = _sc_gather(table, idx2, B * M, Cf)
    sampled_feature = gathered.reshape(B, M, Cf).transpose(0, 2, 1)

    return sampled_xyz, sampled_feature
